# baseline (speedup 1.0000x reference)
import dataclasses
import functools

import jax
import jax.numpy as jnp
from jax import lax
from jax.experimental import pallas as pl
from jax.experimental.pallas import tpu as pltpu
from jax.experimental.pallas import tpu_sc as plsc

N = 50000
C = 128
KV = 27
P = 8192
TOT = KV * P
NC = 2
NS = 16
NW = NC * NS

G_ROWS = TOT // NW
G_CH = 128
G_NCH = G_ROWS // G_CH

TILE = 8352
TPC = 3
S_ROWS = TOT // NS
SEG = 1728
SEG_G = SEG // 16
CCH = 128
CAP = 5248
CAP_CH = CAP // CCH
DUMP = TILE
ACC_ROWS = 8448
WB_CH = 96
WB_N = TILE // WB_CH


def _gather_sc(feats, in_idx):
    mesh = plsc.VectorSubcoreMesh(core_axis_name="c", subcore_axis_name="s")

    @functools.partial(
        pl.kernel,
        out_type=jax.ShapeDtypeStruct((TOT, C), jnp.float32),
        mesh=mesh,
        scratch_types=[
            pltpu.VMEM((G_ROWS,), jnp.int32),
            pltpu.VMEM((G_CH, C), jnp.float32),
            pltpu.VMEM((G_CH, C), jnp.float32),
            pltpu.SemaphoreType.DMA,
            pltpu.SemaphoreType.DMA,
        ],
    )
    def k(feats_hbm, idx_hbm, out_hbm, idx_v, rows_a, rows_b, sem_a, sem_b):
        wid = lax.axis_index("s") * NC + lax.axis_index("c")
        base = wid * G_ROWS
        pltpu.sync_copy(idx_hbm.at[pl.ds(base, G_ROWS)], idx_v)

        @pl.loop(0, G_NCH // 2)
        def _(p):
            j0 = 2 * p
            j1 = j0 + 1
            ca = pltpu.async_copy(
                feats_hbm.at[idx_v.at[pl.ds(j0 * G_CH, G_CH)]], rows_a, sem_a)
            cb = pltpu.async_copy(
                feats_hbm.at[idx_v.at[pl.ds(j1 * G_CH, G_CH)]], rows_b, sem_b)
            ca.wait()
            pltpu.sync_copy(rows_a, out_hbm.at[pl.ds(base + j0 * G_CH, G_CH)])
            cb.wait()
            pltpu.sync_copy(rows_b, out_hbm.at[pl.ds(base + j1 * G_CH, G_CH)])

    return k(feats, in_idx)


def _gemm_tc(gathered, w):
    BP = 4096

    def body(x_ref, w_ref, o_ref):
        x = x_ref[0].astype(jnp.bfloat16)
        wb = w_ref[0].astype(jnp.bfloat16)
        o_ref[...] = jnp.dot(x, wb, preferred_element_type=jnp.float32)[None]

    return pl.pallas_call(
        body,
        grid=(KV, P // BP),
        in_specs=[
            pl.BlockSpec((1, BP, C), lambda k, p: (k, p, 0)),
            pl.BlockSpec((1, C, C), lambda k, p: (k, 0, 0)),
        ],
        out_specs=pl.BlockSpec((1, BP, C), lambda k, p: (k, p, 0)),
        out_shape=jax.ShapeDtypeStruct((KV, P, C), jnp.float32),
        compiler_params=pltpu.CompilerParams(
            dimension_semantics=("parallel", "arbitrary"),
        ),
    )(gathered, w)


def _sc_compiler_params():
    cp = pltpu.CompilerParams()
    if "needs_layout_passes" in pltpu.CompilerParams.__dataclass_fields__:
        cp = dataclasses.replace(cp, needs_layout_passes=False)
    return cp


def _scatter_sc(contrib, out_idx):
    mesh = plsc.VectorSubcoreMesh(core_axis_name="c", subcore_axis_name="s")

    @functools.partial(
        pl.kernel,
        out_type=jax.ShapeDtypeStruct((N, C), jnp.float32),
        mesh=mesh,
        compiler_params=_sc_compiler_params(),
        scratch_types=[
            pltpu.VMEM((SEG,), jnp.int32),
            pltpu.VMEM((CAP_CH, CCH), jnp.int32),
            pltpu.VMEM((CAP_CH, CCH), jnp.int32),
            pltpu.VMEM((CCH, C), jnp.float32),
            pltpu.VMEM((CCH, C), jnp.float32),
            pltpu.VMEM_SHARED((ACC_ROWS, C), jnp.float32),
            pltpu.SemaphoreType.DMA,
            pltpu.SemaphoreType.DMA,
        ],
    )
    def k(contrib_hbm, idx_hbm, out_hbm, idxseg, loc, pos, rows_a, rows_b,
          acc, sem_a, sem_b):
        cid = lax.axis_index("c")
        sid = lax.axis_index("s")

        zero16f = jnp.zeros((16,), jnp.float32)
        zero16i = jnp.zeros((16,), jnp.int32)
        dump16 = jnp.full((16,), DUMP, jnp.int32)
        lane = lax.iota(jnp.int32, 16)

        def process(cnt):
            top = lax.bitwise_and(cnt + CCH - 1, -CCH)
            for gi in range(CCH // 16):
                q = cnt + gi * 16 + lane
                maskp = q < top
                row_i = lax.shift_right_logical(q, 7)
                col_i = lax.bitwise_and(q, CCH - 1)
                plsc.store_scatter(loc, [row_i, col_i], dump16, mask=maskp)
                plsc.store_scatter(pos, [row_i, col_i], zero16i, mask=maskp)

            nch = lax.shift_right_logical(top, 7)

            def chunk_pair(p, carry):
                j0 = 2 * p
                j1 = j0 + 1
                ca = pltpu.async_copy(contrib_hbm.at[pos.at[j0]], rows_a,
                                      sem_a)

                @pl.when(j1 < nch)
                def _():
                    pltpu.async_copy(contrib_hbm.at[pos.at[j1]], rows_b,
                                     sem_b)

                ca.wait()
                pltpu.sync_copy(rows_a, acc.at[loc.at[j0]], add=True)

                @pl.when(j1 < nch)
                def _():
                    pltpu.make_async_copy(contrib_hbm.at[pos.at[j1]], rows_b,
                                          sem_b).wait()
                    pltpu.sync_copy(rows_b, acc.at[loc.at[j1]], add=True)

                return carry

            lax.fori_loop(0, lax.shift_right_logical(nch + 1, 1), chunk_pair,
                          jnp.int32(0))
            return jnp.int32(0)

        for t_local in range(TPC):
            base = (TPC * cid + t_local) * TILE
            rows_t = jnp.minimum(TILE, N - base)

            @pl.loop(0, CCH)
            def _(r):
                @pl.loop(0, C, step=16)
                def _(cc):
                    rows_a[r, pl.ds(cc, 16)] = zero16f

            @pl.loop(0, ACC_ROWS // CCH)
            def _(m):
                @pl.when(lax.rem(m, NS) == sid)
                def _():
                    pltpu.sync_copy(rows_a, acc.at[pl.ds(m * CCH, CCH)])

            plsc.subcore_barrier()

            def seg_body(g, cnt):
                cnt = lax.cond(cnt + SEG > CAP, process,
                               lambda c: c, cnt)
                pltpu.sync_copy(
                    idx_hbm.at[pl.ds(sid * S_ROWS + g * SEG, SEG)], idxseg)

                def scan_group(i, cnt):
                    col = i * 16
                    v = idxseg[pl.ds(col, 16)]
                    localv = v - base
                    maskv = (localv >= 0) & (localv < rows_t)
                    mi = maskv.astype(jnp.int32)
                    pc = plsc.cumsum(mi)
                    q = cnt + pc - 1
                    row_i = lax.shift_right_logical(q, 7)
                    col_i = lax.bitwise_and(q, CCH - 1)
                    plsc.store_scatter(loc, [row_i, col_i], localv,
                                       mask=maskv)
                    pv = (sid * S_ROWS + g * SEG + col) + lane
                    plsc.store_scatter(pos, [row_i, col_i], pv, mask=maskv)
                    return cnt + jnp.sum(mi)

                return lax.fori_loop(0, SEG_G, scan_group, cnt)

            cnt = lax.fori_loop(0, S_ROWS // SEG, seg_body, jnp.int32(0))
            cnt = process(cnt)

            plsc.subcore_barrier()

            mcov = rows_t - lax.rem(rows_t, WB_CH)

            @pl.loop(0, WB_N)
            def _(m):
                @pl.when((lax.rem(m, NS) == sid) & ((m + 1) * WB_CH <= rows_t))
                def _():
                    pltpu.sync_copy(acc.at[pl.ds(m * WB_CH, WB_CH)],
                                    out_hbm.at[pl.ds(base + m * WB_CH, WB_CH)])

            for mt in range(WB_CH // 16):
                @pl.when((sid == mt) & (mcov + (mt + 1) * 16 <= rows_t))
                def _():
                    pltpu.sync_copy(
                        acc.at[pl.ds(mcov + mt * 16, 16)],
                        out_hbm.at[pl.ds(base + mcov + mt * 16, 16)])

            plsc.subcore_barrier()

    return k(contrib, out_idx)


def kernel(coords, feats, maps, mappat, kernel):
    w = kernel
    in_idx = maps[:, :, 0].reshape(TOT)
    out_idx = maps[:, :, 1].reshape(TOT)
    gathered = _gather_sc(feats, in_idx)
    contrib = _gemm_tc(gathered.reshape(KV, P, C), w)
    return _scatter_sc(contrib.reshape(TOT, C), out_idx)
